# baseline (speedup 1.0000x reference)
import functools

import jax
import jax.numpy as jnp
from jax import lax
from jax.experimental import pallas as pl
from jax.experimental.pallas import tpu as pltpu
from jax.experimental.pallas import tpu_sc as plsc

_NC = 2
_NS = 16
_NW = _NC * _NS

_TCH = 200


def _extract_lanes(src, dst, n, dim):
    @pl.loop(0, n, unroll=8)
    def _(j):
        for h in range(dim // 16):
            dst[j, pl.ds(16 * h, 16)] = src[j, pl.ds(16 * h, 16)]


@functools.lru_cache(maxsize=None)
def _build_pad(vocab: int, dim: int):
    n_chunks = vocab // _TCH

    def body(table_hbm, pad_hbm, a0, a1, bufb, sem_a0, sem_a1, sem_w):
        wid = lax.axis_index("s") * _NC + lax.axis_index("c")
        my = (n_chunks - wid + _NW - 1) // _NW

        def r0_of(k):
            return (wid + k * _NW) * _TCH

        def rd_start(k, buf, sem):
            pltpu.async_copy(table_hbm.at[pl.ds(r0_of(k), _TCH)], buf, sem)

        def rd_wait(k, buf, sem):
            pltpu.make_async_copy(table_hbm.at[pl.ds(r0_of(k), _TCH)],
                                  buf, sem).wait()

        def step(k, buf, obuf, sem, osem):
            rd_wait(k, buf, sem)

            @pl.when(k + 1 < my)
            def _():
                rd_start(k + 1, obuf, osem)

            _extract_lanes(buf, bufb, _TCH, dim)
            pltpu.async_copy(bufb, pad_hbm.at[pl.ds(r0_of(k), _TCH)], sem_w)
            pltpu.make_async_copy(bufb, pad_hbm.at[pl.ds(r0_of(k), _TCH)],
                                  sem_w).wait()

        @pl.when(my > 0)
        def _():
            rd_start(0, a0, sem_a0)

            @pl.loop(0, 2 * (my // 2), step=2)
            def _(k):
                step(k, a0, a1, sem_a0, sem_a1)
                step(k + 1, a1, a0, sem_a1, sem_a0)

            @pl.when(my % 2 == 1)
            def _():
                step(my - 1, a0, a1, sem_a0, sem_a1)

    return pl.kernel(
        body,
        out_type=jax.ShapeDtypeStruct((vocab, 128), jnp.float32),
        compiler_params=pltpu.CompilerParams(use_tc_tiling_on_sc=True),
        mesh=plsc.VectorSubcoreMesh(core_axis_name="c", subcore_axis_name="s"),
        scratch_types=[
            pltpu.VMEM((_TCH, dim), jnp.float32),
            pltpu.VMEM((_TCH, dim), jnp.float32),
            pltpu.VMEM((_TCH, 128), jnp.float32),
            pltpu.SemaphoreType.DMA,
            pltpu.SemaphoreType.DMA,
            pltpu.SemaphoreType.DMA,
        ],
    )


@functools.lru_cache(maxsize=None)
def _build_gather(nb: int, seq: int, vocab: int, dim: int):
    per_w = nb // _NW

    def body(x_hbm, pad_hbm, out_hbm, *scratch):
        idx_v = scratch[0:2]
        rows = scratch[2:4]
        obuf = scratch[4:6]
        sem_i = scratch[6:8]
        sem_g = scratch[8:10]
        sem_o = scratch[10:12]

        wid = lax.axis_index("s") * _NC + lax.axis_index("c")
        base = wid * per_w

        def idx_start(i, b):
            pltpu.async_copy(x_hbm.at[base + i], idx_v[b], sem_i[b])

        def idx_wait(i, b):
            pltpu.make_async_copy(x_hbm.at[base + i], idx_v[b],
                                  sem_i[b]).wait()

        def g_start(i, b):
            pltpu.async_copy(pad_hbm.at[idx_v[b]], rows[b], sem_g[b])

        def g_wait(i, b):
            pltpu.make_async_copy(pad_hbm.at[idx_v[b]], rows[b],
                                  sem_g[b]).wait()

        def out_start(i, b):
            pltpu.async_copy(obuf[b], out_hbm.at[base + i], sem_o[b])

        def out_wait(i, b):
            pltpu.make_async_copy(obuf[b], out_hbm.at[base + i],
                                  sem_o[b]).wait()

        def step(i, b, *, first=False, g1=True, i2=True):
            if g1:
                idx_wait(i + 1, 1 - b)
                g_start(i + 1, 1 - b)
            g_wait(i, b)
            if not first:
                out_wait(i - 1, 1 - b)
            _extract_lanes(rows[b], obuf[b], seq, dim)
            out_start(i, b)
            if i2:
                idx_start(i + 2, b)

        idx_start(0, 0)
        idx_start(1, 1)
        idx_wait(0, 0)
        g_start(0, 0)

        step(0, 0, first=True)

        n_main = ((per_w - 3) // 2) * 2

        @pl.loop(1, 1 + n_main, step=2)
        def _(i):
            step(i, 1)
            step(i + 1, 0)

        for i in range(1 + n_main, per_w):
            step(i, i % 2, g1=(i + 1 < per_w), i2=(i + 2 < per_w))

        out_wait(per_w - 1, (per_w - 1) % 2)

    return pl.kernel(
        body,
        out_type=jax.ShapeDtypeStruct((nb, seq, dim), jnp.float32),
        compiler_params=pltpu.CompilerParams(use_tc_tiling_on_sc=True),
        mesh=plsc.VectorSubcoreMesh(core_axis_name="c", subcore_axis_name="s"),
        scratch_types=(
            [pltpu.VMEM((seq,), jnp.int32) for _ in range(2)]
            + [pltpu.VMEM((seq, 128), jnp.float32) for _ in range(2)]
            + [pltpu.VMEM((seq, dim), jnp.float32) for _ in range(2)]
            + [pltpu.SemaphoreType.DMA for _ in range(6)]
        ),
    )


def kernel(x, emb_weight):
    vocab, dim = emb_weight.shape
    nb, seq = x.shape
    slot_table = _build_pad(vocab, dim)(emb_weight)
    out = _build_gather(nb, seq, vocab, dim)(x, slot_table)
    return (out, 0.0)


# speedup vs baseline: 2.3072x; 2.3072x over previous
import functools

import jax
import jax.numpy as jnp
from jax import lax
from jax.experimental import pallas as pl
from jax.experimental.pallas import tpu as pltpu
from jax.experimental.pallas import tpu_sc as plsc

_NC = 2
_NS = 16


@functools.lru_cache(maxsize=None)
def _build(seq: int, nb: int, dim: int, vocab: int):
    half = nb // _NC
    n_full = seq // _NS
    rem = seq - n_full * _NS

    def body(xt_hbm, et_hbm, out_hbm, idx0, idx1, g0, g1, spm,
             sem_i0, sem_i1, sem_g0, sem_g1, sem_w0, sem_w1):
        ci = lax.axis_index("c")
        si = lax.axis_index("s")
        b0 = ci * half
        idxb = (idx0, idx1)
        gb = (g0, g1)
        sem_i = (sem_i0, sem_i1)
        sem_g = (sem_g0, sem_g1)
        sem_w = (sem_w0, sem_w1)

        nr = n_full + jnp.where(si < rem, 1, 0)

        def row_of(k):
            return si + k * _NS

        def idx_start(k, p):
            pltpu.async_copy(xt_hbm.at[row_of(k), pl.ds(b0, half)],
                             idxb[p], sem_i[p])

        def idx_wait(k, p):
            pltpu.make_async_copy(xt_hbm.at[row_of(k), pl.ds(b0, half)],
                                  idxb[p], sem_i[p]).wait()

        def w_start(c, k, p):
            pltpu.async_copy(gb[p], out_hbm.at[row_of(k), c, pl.ds(b0, half)],
                             sem_w[p])

        def w_wait(c, k, p):
            pltpu.make_async_copy(gb[p],
                                  out_hbm.at[row_of(k), c, pl.ds(b0, half)],
                                  sem_w[p]).wait()

        @pl.loop(0, dim)
        def _(c):
            @pl.when(si == 0)
            def _():
                pltpu.sync_copy(et_hbm.at[c], spm)

            plsc.subcore_barrier()

            idx_start(0, 0)
            for k in range(n_full + 1):
                p = k % 2

                @pl.when(k < nr)
                def _():
                    idx_wait(k, p)

                    @pl.when(k + 1 < nr)
                    def _():
                        idx_start(k + 1, 1 - p)

                    pltpu.async_copy(spm.at[idxb[p]], gb[p], sem_g[p])
                    pltpu.make_async_copy(spm.at[idxb[p]], gb[p],
                                          sem_g[p]).wait()

                    @pl.when(k >= 2)
                    def _():
                        w_wait(c, k - 2, p)

                    w_start(c, k, p)

            for k in (n_full - 1, n_full):
                @pl.when(k == nr - 1)
                def _():
                    w_wait(c, k - 1, (k - 1) % 2)
                    w_wait(c, k, k % 2)

            plsc.subcore_barrier()

    return pl.kernel(
        body,
        out_type=jax.ShapeDtypeStruct((seq, dim, nb), jnp.float32),
        compiler_params=pltpu.CompilerParams(use_tc_tiling_on_sc=True),
        mesh=plsc.VectorSubcoreMesh(core_axis_name="c", subcore_axis_name="s"),
        scratch_types=[
            pltpu.VMEM((half,), jnp.int32),
            pltpu.VMEM((half,), jnp.int32),
            pltpu.VMEM((half,), jnp.float32),
            pltpu.VMEM((half,), jnp.float32),
            pltpu.VMEM_SHARED((vocab,), jnp.float32),
            pltpu.SemaphoreType.DMA,
            pltpu.SemaphoreType.DMA,
            pltpu.SemaphoreType.DMA,
            pltpu.SemaphoreType.DMA,
            pltpu.SemaphoreType.DMA,
            pltpu.SemaphoreType.DMA,
        ],
    )


def kernel(x, emb_weight):
    vocab, dim = emb_weight.shape
    nb, seq = x.shape
    xt = x.T
    et = emb_weight.T
    out_t = _build(seq, nb, dim, vocab)(xt, et)
    return (jnp.transpose(out_t, (2, 0, 1)), 0.0)


# speedup vs baseline: 2.3140x; 1.0030x over previous
import functools

import jax
import jax.numpy as jnp
from jax import lax
from jax.experimental import pallas as pl
from jax.experimental.pallas import tpu as pltpu
from jax.experimental.pallas import tpu_sc as plsc

_NC = 2
_NS = 16


@functools.lru_cache(maxsize=None)
def _build(seq: int, nb: int, dim: int, vocab: int):
    half = nb // _NC
    n_full = seq // _NS
    rem = seq - n_full * _NS

    def body(xt_hbm, et_hbm, out_hbm, idx0, idx1, g0, g1, spm,
             sem_i0, sem_i1, sem_g0, sem_g1, sem_w0, sem_w1):
        ci = lax.axis_index("c")
        si = lax.axis_index("s")
        b0 = ci * half
        idxb = (idx0, idx1)
        gb = (g0, g1)
        sem_i = (sem_i0, sem_i1)
        sem_g = (sem_g0, sem_g1)
        sem_w = (sem_w0, sem_w1)

        nr = n_full + jnp.where(si < rem, 1, 0)

        def row_of(k):
            return si + k * _NS

        def idx_start(k, p):
            pltpu.async_copy(xt_hbm.at[row_of(k), pl.ds(b0, half)],
                             idxb[p], sem_i[p])

        def idx_wait(k, p):
            pltpu.make_async_copy(xt_hbm.at[row_of(k), pl.ds(b0, half)],
                                  idxb[p], sem_i[p]).wait()

        def w_start(c, k, p):
            pltpu.async_copy(gb[p], out_hbm.at[row_of(k), c, pl.ds(b0, half)],
                             sem_w[p])

        def w_wait(c, k, p):
            pltpu.make_async_copy(gb[p],
                                  out_hbm.at[row_of(k), c, pl.ds(b0, half)],
                                  sem_w[p]).wait()

        @pl.loop(0, dim)
        def _(c):
            @pl.when(si == 0)
            def _():
                pltpu.sync_copy(et_hbm.at[c], spm)

            plsc.subcore_barrier()

            idx_start(0, 0)
            for k in range(n_full + 1):
                p = k % 2

                @pl.when(k < nr)
                def _():
                    idx_wait(k, p)

                    @pl.when(k + 1 < nr)
                    def _():
                        idx_start(k + 1, 1 - p)

                    q4 = half // 4
                    for q in range(4):
                        pltpu.async_copy(
                            spm.at[idxb[p].at[pl.ds(q * q4, q4)]],
                            gb[p].at[pl.ds(q * q4, q4)], sem_g[q % 2])
                    for q in range(4):
                        pltpu.make_async_copy(
                            spm.at[idxb[p].at[pl.ds(q * q4, q4)]],
                            gb[p].at[pl.ds(q * q4, q4)], sem_g[q % 2]).wait()

                    @pl.when(k >= 2)
                    def _():
                        w_wait(c, k - 2, p)

                    w_start(c, k, p)

            for k in (n_full - 1, n_full):
                @pl.when(k == nr - 1)
                def _():
                    w_wait(c, k - 1, (k - 1) % 2)
                    w_wait(c, k, k % 2)

            plsc.subcore_barrier()

    return pl.kernel(
        body,
        out_type=jax.ShapeDtypeStruct((seq, dim, nb), jnp.float32),
        compiler_params=pltpu.CompilerParams(use_tc_tiling_on_sc=True),
        mesh=plsc.VectorSubcoreMesh(core_axis_name="c", subcore_axis_name="s"),
        scratch_types=[
            pltpu.VMEM((half,), jnp.int32),
            pltpu.VMEM((half,), jnp.int32),
            pltpu.VMEM((half,), jnp.float32),
            pltpu.VMEM((half,), jnp.float32),
            pltpu.VMEM_SHARED((vocab,), jnp.float32),
            pltpu.SemaphoreType.DMA,
            pltpu.SemaphoreType.DMA,
            pltpu.SemaphoreType.DMA,
            pltpu.SemaphoreType.DMA,
            pltpu.SemaphoreType.DMA,
            pltpu.SemaphoreType.DMA,
        ],
    )


def kernel(x, emb_weight):
    vocab, dim = emb_weight.shape
    nb, seq = x.shape
    xt = x.T
    et = emb_weight.T
    out_t = _build(seq, nb, dim, vocab)(xt, et)
    return (jnp.transpose(out_t, (2, 0, 1)), 0.0)
